# baseline (speedup 1.0000x reference)
import functools
import jax
import jax.numpy as jnp
from jax import lax
from jax.experimental import pallas as pl
from jax.experimental.pallas import tpu as pltpu, tpu_sc as plsc


def kernel(tokens, embedding_weight):
    seq_len = tokens.shape[1]
    _, d_model = embedding_weight.shape
    nw = 32
    rows_per_w = seq_len // nw
    mesh = plsc.VectorSubcoreMesh(core_axis_name="c", subcore_axis_name="s")

    @functools.partial(
        pl.kernel,
        mesh=mesh,
        out_type=jax.ShapeDtypeStruct((seq_len, d_model), jnp.float32),
    )
    def k(table_hbm, out_hbm):
        wid = lax.axis_index("s") * 2 + lax.axis_index("c")
        base = wid * rows_per_w
        pltpu.sync_copy(
            table_hbm.at[pl.ds(base, rows_per_w)],
            out_hbm.at[pl.ds(base, rows_per_w)],
        )

    return k(embedding_weight)[None]


# speedup vs baseline: 31.3695x; 31.3695x over previous
import functools
import jax
import jax.numpy as jnp
from jax import lax
from jax.experimental import pallas as pl
from jax.experimental.pallas import tpu as pltpu, tpu_sc as plsc

_NW = 32
_CHUNK = 16


def kernel(tokens, embedding_weight):
    seq_len = tokens.shape[1]
    _, d_model = embedding_weight.shape
    rows_per_w = seq_len // _NW
    nchunk = rows_per_w // _CHUNK
    mesh = plsc.VectorSubcoreMesh(core_axis_name="c", subcore_axis_name="s")

    @functools.partial(
        pl.kernel,
        mesh=mesh,
        out_type=jax.ShapeDtypeStruct((seq_len, d_model), jnp.float32),
        scratch_types=[
            pltpu.VMEM((_CHUNK, d_model), jnp.float32),
            pltpu.VMEM((_CHUNK, d_model), jnp.float32),
            pltpu.SemaphoreType.DMA,
            pltpu.SemaphoreType.DMA,
            pltpu.SemaphoreType.DMA,
            pltpu.SemaphoreType.DMA,
        ],
    )
    def k(table, out, buf0, buf1, l0, l1, s0, s1):
        wid = lax.axis_index("s") * 2 + lax.axis_index("c")
        base = wid * rows_per_w
        bufs = (buf0, buf1)
        lsems = (l0, l1)
        ssems = (s0, s1)

        def load(c):
            return pltpu.make_async_copy(
                table.at[pl.ds(base + c * _CHUNK, _CHUNK)], bufs[c % 2], lsems[c % 2]
            )

        def store(c):
            return pltpu.make_async_copy(
                bufs[c % 2], out.at[pl.ds(base + c * _CHUNK, _CHUNK)], ssems[c % 2]
            )

        load(0).start()
        for c in range(nchunk):
            if c + 1 < nchunk:
                if c >= 1:
                    store(c - 1).wait()
                load(c + 1).start()
            load(c).wait()
            store(c).start()
        store(nchunk - 2).wait()
        store(nchunk - 1).wait()

    return k(embedding_weight)[None]


# speedup vs baseline: 35.7716x; 1.1403x over previous
import jax
import jax.numpy as jnp
from jax.experimental import pallas as pl
from jax.experimental.pallas import tpu as pltpu

_CHUNK = 512
_NBUF = 4


def _dma_ring(in_ref, out_ref, *rest):
    bufs = rest[:_NBUF]
    lsems = rest[_NBUF : 2 * _NBUF]
    ssems = rest[2 * _NBUF :]
    rows = in_ref.shape[0]
    nchunk = rows // _CHUNK

    def load(c):
        b = c % _NBUF
        return pltpu.make_async_copy(
            in_ref.at[pl.ds(c * _CHUNK, _CHUNK)], bufs[b], lsems[b]
        )

    def store(c):
        b = c % _NBUF
        return pltpu.make_async_copy(
            bufs[b], out_ref.at[pl.ds(c * _CHUNK, _CHUNK)], ssems[b]
        )

    for c in range(min(_NBUF, nchunk)):
        load(c).start()
    for c in range(nchunk):
        if c >= _NBUF:
            store(c - _NBUF).wait()
            load(c).start()
        load(c).wait()
        store(c).start()
    for c in range(max(nchunk - _NBUF, 0), nchunk):
        store(c).wait()


def kernel(tokens, embedding_weight):
    seq_len = tokens.shape[1]
    _, d_model = embedding_weight.shape
    out = pl.pallas_call(
        _dma_ring,
        in_specs=[pl.BlockSpec(memory_space=pl.ANY)],
        out_specs=pl.BlockSpec(memory_space=pl.ANY),
        scratch_shapes=(
            [pltpu.VMEM((_CHUNK, d_model), jnp.float32)] * _NBUF
            + [pltpu.SemaphoreType.DMA] * (2 * _NBUF)
        ),
        out_shape=jax.ShapeDtypeStruct((seq_len, d_model), embedding_weight.dtype),
    )(embedding_weight)
    return out[None]
